# baseline (speedup 1.0000x reference)
import jax
import jax.numpy as jnp
from jax.experimental import pallas as pl
from jax.experimental.pallas import tpu as pltpu


def _round_up(a, b):
    return (a + b - 1) // b * b


def _convtr_kernel(xp_ref, wa_ref, wb_ref, b_ref, o_ref, nlc_ref):
    q = nlc_ref.shape[1] // 8
    c_out = o_ref.shape[0]
    x_q = xp_ref[1:q + 1, :]
    a = jnp.concatenate([x_q, xp_ref[0:q, :]], axis=1)
    b = jnp.concatenate([xp_ref[2:q + 2, :], x_q], axis=1)
    bias = b_ref[...]
    lo = jnp.dot(a, wa_ref[...], preferred_element_type=jnp.float32) + bias
    hi = jnp.dot(b, wb_ref[...], preferred_element_type=jnp.float32) + bias
    n_half = c_out // 128
    for h in range(n_half):
        for p in range(4):
            c0 = p * c_out + h * 128
            nlc_ref[h, p:p + 8 * q:8, :] = lo[:, c0:c0 + 128]
            nlc_ref[h, p + 4:p + 4 + 8 * q:8, :] = hi[:, c0:c0 + 128]
        o_ref[h * 128:(h + 1) * 128, :] = jnp.transpose(nlc_ref[h], (1, 0))


def kernel(v, g, bias, x):
    c_in, c_out, k = v.shape
    n, _, l_in = x.shape
    s, pad = 8, 4
    l_out = (l_in - 1) * s - 2 * pad + k
    q_len = -(-l_out // s)

    scale = (g[:, 0, 0] / jnp.sqrt(jnp.sum(v * v, axis=(1, 2))))

    def taps(lo, hi):
        return v[:, :, lo:hi].transpose(0, 2, 1).reshape(c_in, (hi - lo) * c_out)

    wa = jnp.concatenate([taps(4, 8), taps(12, 16)], axis=0).astype(jnp.bfloat16)
    wb = jnp.concatenate([taps(0, 4), taps(8, 12)], axis=0).astype(jnp.bfloat16)
    bias_row = jnp.tile(bias.astype(jnp.float32), (4,))[None, :]

    l_pad = _round_up(q_len + 2, 8)
    xp = jnp.pad(x.transpose(0, 2, 1) * scale[None, None, :],
                 ((0, 0), (1, l_pad - l_in - 1), (0, 0))).astype(jnp.bfloat16)

    out = pl.pallas_call(
        _convtr_kernel,
        out_shape=jax.ShapeDtypeStruct((n, c_out, q_len * s), jnp.float32),
        grid=(n,),
        in_specs=[
            pl.BlockSpec((None, l_pad, c_in), lambda b: (b, 0, 0)),
            pl.BlockSpec((2 * c_in, 4 * c_out), lambda b: (0, 0)),
            pl.BlockSpec((2 * c_in, 4 * c_out), lambda b: (0, 0)),
            pl.BlockSpec((1, 4 * c_out), lambda b: (0, 0)),
        ],
        out_specs=pl.BlockSpec((None, c_out, q_len * s), lambda b: (b, 0, 0)),
        scratch_shapes=[pltpu.VMEM((c_out // 128, q_len * s, 128), jnp.float32)],
        compiler_params=pltpu.CompilerParams(
            dimension_semantics=("parallel",)),
    )(xp, wa, wb, bias_row)

    return out[:, :, :l_out]


# speedup vs baseline: 1.1235x; 1.1235x over previous
import jax
import jax.numpy as jnp
from jax.experimental import pallas as pl
from jax.experimental.pallas import tpu as pltpu


def _round_up(a, b):
    return (a + b - 1) // b * b


def _convtr_kernel(xa_ref, wa_ref, wb_ref, b_ref, o_ref, nlc_ref):
    q = nlc_ref.shape[1] // 8
    c_out = o_ref.shape[0]
    a = xa_ref[0:q, :]
    b = xa_ref[1:q + 1, :]
    bias = b_ref[...]
    lo = jnp.dot(a, wa_ref[...], preferred_element_type=jnp.float32) + bias
    hi = jnp.dot(b, wb_ref[...], preferred_element_type=jnp.float32) + bias
    n_half = c_out // 128
    for h in range(n_half):
        for p in range(4):
            c0 = p * c_out + h * 128
            nlc_ref[h, p:p + 8 * q:8, :] = lo[:, c0:c0 + 128]
            nlc_ref[h, p + 4:p + 4 + 8 * q:8, :] = hi[:, c0:c0 + 128]
        o_ref[h * 128:(h + 1) * 128, :] = jnp.transpose(nlc_ref[h], (1, 0))


def kernel(v, g, bias, x):
    c_in, c_out, k = v.shape
    n, _, l_in = x.shape
    s, pad = 8, 4
    l_out = (l_in - 1) * s - 2 * pad + k
    q_len = -(-l_out // s)

    norm = jnp.sqrt(jnp.sum(v * v, axis=(1, 2), keepdims=True))
    wt = (g * v / norm).transpose(0, 2, 1)

    def taps(lo, hi):
        return wt[:, lo:hi, :].reshape(c_in, (hi - lo) * c_out)

    wa = jnp.concatenate([taps(4, 8), taps(12, 16)], axis=0).astype(jnp.bfloat16)
    wb = jnp.concatenate([taps(0, 4), taps(8, 12)], axis=0).astype(jnp.bfloat16)
    bias_row = jnp.tile(bias.astype(jnp.float32), (4,))[None, :]

    l_pad = _round_up(q_len + 1, 8)
    xt = x.transpose(0, 2, 1).astype(jnp.bfloat16)
    xa = jnp.concatenate(
        [jnp.pad(xt, ((0, 0), (0, l_pad - l_in), (0, 0))),
         jnp.pad(xt, ((0, 0), (1, l_pad - l_in - 1), (0, 0)))],
        axis=2)

    out = pl.pallas_call(
        _convtr_kernel,
        out_shape=jax.ShapeDtypeStruct((n, c_out, q_len * s), jnp.float32),
        grid=(n,),
        in_specs=[
            pl.BlockSpec((None, l_pad, 2 * c_in), lambda b: (b, 0, 0)),
            pl.BlockSpec((2 * c_in, 4 * c_out), lambda b: (0, 0)),
            pl.BlockSpec((2 * c_in, 4 * c_out), lambda b: (0, 0)),
            pl.BlockSpec((1, 4 * c_out), lambda b: (0, 0)),
        ],
        out_specs=pl.BlockSpec((None, c_out, q_len * s), lambda b: (b, 0, 0)),
        scratch_shapes=[pltpu.VMEM((c_out // 128, q_len * s, 128), jnp.float32)],
        compiler_params=pltpu.CompilerParams(
            dimension_semantics=("parallel",)),
    )(xa, wa, wb, bias_row)

    return out[:, :, :l_out]


# speedup vs baseline: 1.1712x; 1.0425x over previous
import jax
import jax.numpy as jnp
from jax.experimental import pallas as pl
from jax.experimental.pallas import tpu as pltpu


def _round_up(a, b):
    return (a + b - 1) // b * b


def _wprep_kernel(w_ref, g_ref, bias_ref, wa_ref, wb_ref, brow_ref):
    c_in, k, c_out = w_ref.shape
    w = w_ref[...]
    scale = g_ref[...] * jax.lax.rsqrt(jnp.sum(w * w, axis=(1, 2)))[:, None]
    ws = (w * scale[:, :, None]).astype(jnp.bfloat16)
    for p in range(4):
        wa_ref[0:c_in, p * c_out:(p + 1) * c_out] = ws[:, p + 4, :]
        wa_ref[c_in:2 * c_in, p * c_out:(p + 1) * c_out] = ws[:, p + 12, :]
        wb_ref[0:c_in, p * c_out:(p + 1) * c_out] = ws[:, p, :]
        wb_ref[c_in:2 * c_in, p * c_out:(p + 1) * c_out] = ws[:, p + 8, :]
    brow_ref[...] = jnp.concatenate([bias_ref[...]] * 4, axis=1)


def _convtr_kernel(xp_ref, wa_ref, wb_ref, b_ref, o_ref, nlc_ref):
    q = nlc_ref.shape[1] // 8
    c_out = o_ref.shape[0]
    x_q = xp_ref[1:q + 1, :]
    a = jnp.concatenate([x_q, xp_ref[0:q, :]], axis=1)
    b = jnp.concatenate([xp_ref[2:q + 2, :], x_q], axis=1)
    bias = b_ref[...]
    lo = jnp.dot(a, wa_ref[...], preferred_element_type=jnp.float32) + bias
    hi = jnp.dot(b, wb_ref[...], preferred_element_type=jnp.float32) + bias
    n_half = c_out // 128
    for h in range(n_half):
        for p in range(4):
            c0 = p * c_out + h * 128
            nlc_ref[h, p:p + 8 * q:8, :] = lo[:, c0:c0 + 128]
            nlc_ref[h, p + 4:p + 4 + 8 * q:8, :] = hi[:, c0:c0 + 128]
        o_ref[h * 128:(h + 1) * 128, :] = jnp.transpose(nlc_ref[h], (1, 0))


def kernel(v, g, bias, x):
    c_in, c_out, k = v.shape
    n, _, l_in = x.shape
    s, pad = 8, 4
    l_out = (l_in - 1) * s - 2 * pad + k
    q_len = -(-l_out // s)

    wa, wb, bias_row = pl.pallas_call(
        _wprep_kernel,
        out_shape=(
            jax.ShapeDtypeStruct((2 * c_in, 4 * c_out), jnp.bfloat16),
            jax.ShapeDtypeStruct((2 * c_in, 4 * c_out), jnp.bfloat16),
            jax.ShapeDtypeStruct((1, 4 * c_out), jnp.float32),
        ),
    )(v.transpose(0, 2, 1), g.reshape(c_in, 1),
      bias.astype(jnp.float32)[None, :])

    l_pad = _round_up(q_len + 2, 8)
    xp = jnp.pad(x.transpose(0, 2, 1),
                 ((0, 0), (1, l_pad - l_in - 1), (0, 0))).astype(jnp.bfloat16)

    out = pl.pallas_call(
        _convtr_kernel,
        out_shape=jax.ShapeDtypeStruct((n, c_out, q_len * s), jnp.float32),
        grid=(n,),
        in_specs=[
            pl.BlockSpec((None, l_pad, c_in), lambda b: (b, 0, 0)),
            pl.BlockSpec((2 * c_in, 4 * c_out), lambda b: (0, 0)),
            pl.BlockSpec((2 * c_in, 4 * c_out), lambda b: (0, 0)),
            pl.BlockSpec((1, 4 * c_out), lambda b: (0, 0)),
        ],
        out_specs=pl.BlockSpec((None, c_out, q_len * s), lambda b: (b, 0, 0)),
        scratch_shapes=[pltpu.VMEM((c_out // 128, q_len * s, 128), jnp.float32)],
        compiler_params=pltpu.CompilerParams(
            dimension_semantics=("parallel",)),
    )(xp, wa, wb, bias_row)

    return out[:, :, :l_out]
